# baseline (speedup 1.0000x reference)
import functools

import jax
import jax.numpy as jnp
from jax import lax
from jax.experimental import pallas as pl
from jax.experimental.pallas import tpu as pltpu
from jax.experimental.pallas import tpu_sc as plsc

_NC, _NS = 2, 16
_NW = _NC * _NS
_SR = 16

_NBUF = 5
_PF = 3


def _sc_body(B, S, D, x_hbm, pe_hbm, out_hbm, *scratch):
    wid = lax.axis_index("s") * _NC + lax.axis_index("c")
    rows_w = S // _NW
    nsr = rows_w // _SR
    nch = nsr * B
    clen = _SR * D
    base = wid * rows_w * D
    vx = scratch[:_NBUF]
    vp = scratch[_NBUF:_NBUF + 2]
    isem = scratch[_NBUF + 2:2 * _NBUF + 2]
    osem = scratch[2 * _NBUF + 2:3 * _NBUF + 2]
    psem = scratch[3 * _NBUF + 2:3 * _NBUF + 4]

    def x_copy(ci):
        sr, b = divmod(ci, B)
        buf = ci % _NBUF
        return pltpu.make_async_copy(
            x_hbm.at[b, pl.ds(base + sr * clen, clen)], vx[buf], isem[buf])

    def o_copy(ci):
        sr, b = divmod(ci, B)
        buf = ci % _NBUF
        return pltpu.make_async_copy(
            vx[buf], out_hbm.at[b, pl.ds(base + sr * clen, clen)], osem[buf])

    def pe_copy(sr):
        return pltpu.make_async_copy(
            pe_hbm.at[pl.ds(base + sr * clen, clen)], vp[sr % 2],
            psem[sr % 2])

    pe_copy(0).start()
    for ci in range(_PF):
        x_copy(ci).start()
    for ci in range(nch):
        sr, b = divmod(ci, B)
        if b == 0:
            pe_copy(sr).wait()
            if sr + 1 < nsr:
                pe_copy(sr + 1).start()
        x_copy(ci).wait()

        vx_c = vx[ci % _NBUF]
        vp_c = vp[sr % 2]

        @plsc.parallel_loop(0, clen, step=16, unroll=8)
        def _(i):
            plsc.addupdate(vx_c.at[pl.ds(i, 16)], vp_c[pl.ds(i, 16)])

        o_copy(ci).start()
        nxt = ci + _PF
        if nxt < nch:
            prev = nxt - _NBUF
            if prev >= 0:
                o_copy(prev).wait()
            x_copy(nxt).start()

    for ci in range(max(0, nch - _NBUF), nch):
        o_copy(ci).wait()


def kernel(x, pe):
    B, S, D = x.shape
    mesh = plsc.VectorSubcoreMesh(core_axis_name="c", subcore_axis_name="s")
    k = pl.kernel(
        functools.partial(_sc_body, B, S, D),
        out_type=jax.ShapeDtypeStruct((B, S * D), jnp.float32),
        mesh=mesh,
        scratch_types=(
            [pltpu.VMEM((_SR * D,), jnp.float32)] * (_NBUF + 2)
            + [pltpu.SemaphoreType.DMA] * (_NBUF * 2 + 2)
        ),
    )
    out = k(x.reshape(B, S * D), pe[:S].reshape(S * D))
    return out.reshape(B, S, D)


# speedup vs baseline: 2.5263x; 2.5263x over previous
import functools

import jax
import jax.numpy as jnp
from jax import lax
from jax.experimental import pallas as pl
from jax.experimental.pallas import tpu as pltpu
from jax.experimental.pallas import tpu_sc as plsc

_NC, _NS = 2, 16
_NW = _NC * _NS
_SR = 16

_NBUF = 5
_PF = 3


def _sc_body(B, S, D, x_hbm, pe_hbm, out_hbm, *scratch):
    wid = lax.axis_index("s") * _NC + lax.axis_index("c")
    rows_w = S // _NW
    nsr = rows_w // _SR
    nch = nsr * B
    row_base = wid * rows_w
    vx = scratch[:_NBUF]
    vp = scratch[_NBUF:_NBUF + 2]
    isem = scratch[_NBUF + 2:2 * _NBUF + 2]
    osem = scratch[2 * _NBUF + 2:3 * _NBUF + 2]
    psem = scratch[3 * _NBUF + 2:3 * _NBUF + 4]

    def x_copy(ci):
        sr, b = divmod(ci, B)
        buf = ci % _NBUF
        return pltpu.make_async_copy(
            x_hbm.at[b, pl.ds(row_base + sr * _SR, _SR), :], vx[buf],
            isem[buf])

    def o_copy(ci):
        sr, b = divmod(ci, B)
        buf = ci % _NBUF
        return pltpu.make_async_copy(
            vx[buf], out_hbm.at[b, pl.ds(row_base + sr * _SR, _SR), :],
            osem[buf])

    def pe_copy(sr):
        return pltpu.make_async_copy(
            pe_hbm.at[pl.ds(row_base + sr * _SR, _SR), :], vp[sr % 2],
            psem[sr % 2])

    pe_copy(0).start()
    for ci in range(_PF):
        x_copy(ci).start()
    for ci in range(nch):
        sr, b = divmod(ci, B)
        if b == 0:
            pe_copy(sr).wait()
            if sr + 1 < nsr:
                pe_copy(sr + 1).start()
        x_copy(ci).wait()

        vx_c = vx[ci % _NBUF]
        vp_c = vp[sr % 2]

        @plsc.parallel_loop(0, _SR, step=1)
        def _(r):
            @plsc.parallel_loop(0, D, step=16, unroll=8)
            def _(c):
                plsc.addupdate(vx_c.at[r, pl.ds(c, 16)],
                               vp_c[r, pl.ds(c, 16)])

        o_copy(ci).start()
        nxt = ci + _PF
        if nxt < nch:
            prev = nxt - _NBUF
            if prev >= 0:
                o_copy(prev).wait()
            x_copy(nxt).start()

    for ci in range(max(0, nch - _NBUF), nch):
        o_copy(ci).wait()


def kernel(x, pe):
    B, S, D = x.shape
    mesh = plsc.VectorSubcoreMesh(core_axis_name="c", subcore_axis_name="s")
    k = pl.kernel(
        functools.partial(_sc_body, B, S, D),
        out_type=jax.ShapeDtypeStruct((B, S, D), jnp.float32),
        mesh=mesh,
        scratch_types=(
            [pltpu.VMEM((_SR, D), jnp.float32)] * (_NBUF + 2)
            + [pltpu.SemaphoreType.DMA] * (_NBUF * 2 + 2)
        ),
    )
    return k(x, pe[:S])
